# baseline (speedup 1.0000x reference)
import functools

import jax
import jax.numpy as jnp
from jax import lax
from jax.experimental import pallas as pl
from jax.experimental.pallas import tpu as pltpu
from jax.experimental.pallas import tpu_sc as plsc

NUM_CORES = 2
NUM_SUBCORES = 16
NUM_WORKERS = NUM_CORES * NUM_SUBCORES
DPAD = 128
BB = 128
LANES = 16


def _embed_kernel(row_len, d,
                  table_hbm, x_hbm, out_hbm,
                  idx_v, idx_t, rows0, rows1, obuf0, obuf1,
                  gsem0, gsem1, wsem0, wsem1):
    cid = lax.axis_index("c")
    sid = lax.axis_index("s")
    wid = sid * NUM_CORES + cid
    nfo = d // 8

    pltpu.sync_copy(x_hbm.at[pl.ds(wid * BB, BB)], idx_v)

    bvecs = [lax.iota(jnp.int32, LANES) + (LANES * k)
             for k in range(BB // LANES)]

    def tr_idx(r, carry):
        rfull = jnp.full((LANES,), r, dtype=jnp.int32)
        for k in range(BB // LANES):
            v = plsc.load_gather(idx_v, [bvecs[k], rfull])
            idx_t[r, pl.ds(LANES * k, LANES)] = v
        return carry

    lax.fori_loop(0, row_len, tr_idx, 0)

    def fire_g(r, rows, sem):
        pltpu.async_copy(table_hbm.at[idx_t.at[r]], rows, sem)

    def drain_g(r, rows, sem):
        pltpu.make_async_copy(table_hbm.at[idx_t.at[r]], rows, sem).wait()

    def transpose_chunk(rows, obuf):
        def body(f, carry):
            ffull = jnp.full((LANES,), f, dtype=jnp.int32)
            for k in range(BB // LANES):
                v = plsc.load_gather(rows, [bvecs[k], ffull])
                obuf[f, pl.ds(LANES * k, LANES)] = v
            return carry
        lax.fori_loop(0, d, body, 0)

    def fire_w(r, obuf, sem):
        for fo in range(nfo):
            pltpu.async_copy(obuf.at[pl.ds(fo * 8, 8)],
                             out_hbm.at[r, fo, wid], sem)

    def drain_w(r, obuf, sem):
        for fo in range(nfo):
            pltpu.make_async_copy(obuf.at[pl.ds(fo * 8, 8)],
                                  out_hbm.at[r, fo, wid], sem).wait()

    fire_g(0, rows0, gsem0)

    def body(i, carry):
        r0 = 2 * i
        r1 = r0 + 1
        fire_g(r1, rows1, gsem1)
        drain_g(r0, rows0, gsem0)

        @pl.when(i > 0)
        def _():
            drain_w(r0, obuf0, wsem0)

        transpose_chunk(rows0, obuf0)
        fire_w(r0, obuf0, wsem0)

        @pl.when(r0 + 2 < row_len)
        def _():
            fire_g(r0 + 2, rows0, gsem0)

        drain_g(r1, rows1, gsem1)

        @pl.when(i > 0)
        def _():
            drain_w(r1, obuf1, wsem1)

        transpose_chunk(rows1, obuf1)
        fire_w(r1, obuf1, wsem1)
        return carry

    lax.fori_loop(0, row_len // 2, body, 0)

    drain_w(0, obuf0, wsem0)
    drain_w(1, obuf1, wsem1)


def kernel(x, table):
    n_rows, row_len = x.shape
    d = table.shape[1]
    xi = x.astype(jnp.int32) * 2
    tlin = jnp.pad(table, ((0, 0), (0, DPAD - d))).reshape(
        2 * table.shape[0], d)
    assert n_rows == NUM_WORKERS * BB and row_len % 2 == 0 and d % 8 == 0

    mesh = plsc.VectorSubcoreMesh(core_axis_name="c", subcore_axis_name="s")
    out = pl.kernel(
        functools.partial(_embed_kernel, row_len, d),
        out_type=jax.ShapeDtypeStruct((row_len, d // 8, NUM_WORKERS, 8, BB),
                                      jnp.float32),
        mesh=mesh,
        scratch_types=[
            pltpu.VMEM((BB, row_len), jnp.int32),
            pltpu.VMEM((row_len, BB), jnp.int32),
            pltpu.VMEM((BB, d), jnp.float32),
            pltpu.VMEM((BB, d), jnp.float32),
            pltpu.VMEM((d, BB), jnp.float32),
            pltpu.VMEM((d, BB), jnp.float32),
        ] + [pltpu.SemaphoreType.DMA] * 4,
        compiler_params=pltpu.CompilerParams(use_tc_tiling_on_sc=False,
                                             needs_layout_passes=False),
    )(tlin, xi)
    return out.transpose(2, 4, 0, 1, 3).reshape(n_rows, row_len, d)


# speedup vs baseline: 2.0476x; 2.0476x over previous
import functools

import jax
import jax.numpy as jnp
from jax import lax
from jax.experimental import layout as jex_layout
from jax.experimental import pallas as pl
from jax.experimental.pallas import tpu as pltpu
from jax.experimental.pallas import tpu_sc as plsc

NUM_CORES = 2
NUM_SUBCORES = 16
NUM_WORKERS = NUM_CORES * NUM_SUBCORES
ROWS_PER_GROUP = 2
SPLITS = ((0, 128), (128, 72))
K = ROWS_PER_GROUP * len(SPLITS)
DPAD = 128


def _embed_kernel(rows_per_worker, row_len, d,
                  table_hbm, x_hbm, out_hbm,
                  idx_v, a0, a1, a2, a3, b0, b1, b2, b3,
                  gsem_a, gsem_b, wsem_a, wsem_b):
    cid = lax.axis_index("c")
    sid = lax.axis_index("s")
    wid = sid * NUM_CORES + cid
    row0 = wid * rows_per_worker
    flat0 = row0 * row_len
    bufs_a = (a0, a1, a2, a3)
    bufs_b = (b0, b1, b2, b3)
    n_groups = rows_per_worker // ROWS_PER_GROUP

    pltpu.sync_copy(x_hbm.at[pl.ds(row0, rows_per_worker)], idx_v)

    def chunk_refs(g, j, bufs):
        r = g * ROWS_PER_GROUP + j // len(SPLITS)
        off, sz = SPLITS[j % len(SPLITS)]
        src = table_hbm.at[idx_v.at[r, pl.ds(off, sz)]]
        dst = out_hbm.at[pl.ds(flat0 + r * row_len + off, sz), pl.ds(0, d)]
        return src, bufs[j], dst

    def fire_g(g, bufs, sem):
        for j in range(K):
            src, buf, _ = chunk_refs(g, j, bufs)
            pltpu.async_copy(src, buf, sem)

    def drain_g(g, bufs, sem):
        for j in range(K):
            src, buf, _ = chunk_refs(g, j, bufs)
            pltpu.make_async_copy(src, buf, sem).wait()

    def fire_w(g, bufs, sem):
        for j in range(K):
            _, buf, dst = chunk_refs(g, j, bufs)
            pltpu.async_copy(buf, dst, sem)

    def drain_w(g, bufs, sem):
        for j in range(K):
            _, buf, dst = chunk_refs(g, j, bufs)
            pltpu.make_async_copy(buf, dst, sem).wait()

    fire_g(0, bufs_a, gsem_a)
    drain_g(0, bufs_a, gsem_a)
    fire_w(0, bufs_a, wsem_a)
    fire_g(1, bufs_b, gsem_b)

    def body(i, carry):
        g0 = 2 * i + 1
        g1 = g0 + 1
        drain_g(g0, bufs_b, gsem_b)
        fire_w(g0, bufs_b, wsem_b)
        drain_w(g0 - 1, bufs_a, wsem_a)
        fire_g(g1, bufs_a, gsem_a)
        drain_g(g1, bufs_a, gsem_a)
        fire_w(g1, bufs_a, wsem_a)
        drain_w(g1 - 1, bufs_b, wsem_b)
        fire_g(g1 + 1, bufs_b, gsem_b)
        return carry

    lax.fori_loop(0, (n_groups - 2) // 2, body, 0)

    g_last = n_groups - 1
    drain_g(g_last, bufs_b, gsem_b)
    fire_w(g_last, bufs_b, wsem_b)
    drain_w(g_last - 1, bufs_a, wsem_a)
    drain_w(g_last, bufs_b, wsem_b)


def kernel(x, table):
    n_rows, row_len = x.shape
    d = table.shape[1]
    xi = x.astype(jnp.int32)
    tlin = jnp.pad(table, ((0, 0), (0, DPAD - d))).reshape(2 * table.shape[0], d)
    xi = xi * 2
    rows_per_worker = n_rows // NUM_WORKERS
    b_total = n_rows * row_len
    assert n_rows == NUM_WORKERS * rows_per_worker
    assert rows_per_worker % (2 * ROWS_PER_GROUP) == 0
    assert sum(sz for _, sz in SPLITS) == row_len

    mesh = plsc.VectorSubcoreMesh(core_axis_name="c", subcore_axis_name="s")
    buf_types = [pltpu.VMEM((sz, d), jnp.float32)
                 for _ in range(ROWS_PER_GROUP) for _, sz in SPLITS]
    out = pl.kernel(
        functools.partial(_embed_kernel, rows_per_worker, row_len, d),
        out_type=jax.ShapeDtypeStruct((b_total, DPAD), jnp.float32),
        mesh=mesh,
        scratch_types=[pltpu.VMEM((rows_per_worker, row_len), jnp.int32)]
        + buf_types + buf_types
        + [pltpu.SemaphoreType.DMA] * 4,
        compiler_params=pltpu.CompilerParams(use_tc_tiling_on_sc=False),
    )(tlin, xi)
    return out.reshape(n_rows, row_len, DPAD)[:, :, :d]
